# baseline (speedup 1.0000x reference)
import functools

import numpy as np

import jax
import jax.numpy as jnp
from jax import lax
from jax.experimental import pallas as pl
from jax.experimental.pallas import tpu as pltpu
from jax.experimental.pallas import tpu_sc as plsc

N = 10000
E = 160000
D = 256
H = 128
NT = 16
EPT = E // NT
CH = 64
NCHP = 160
PH = 16
NP = N // 2
ACCR = 5008
ZPT = 312
RS = 624

_PERM128 = np.concatenate(
    [np.concatenate([np.arange(g, g + 32, 2), np.arange(g + 1, g + 32, 2)])
     for g in range(0, H, 32)])
_PERM256 = np.concatenate([_PERM128, H + _PERM128])

_mesh = plsc.VectorSubcoreMesh(core_axis_name="c", subcore_axis_name="s")


@functools.partial(
    pl.kernel,
    mesh=_mesh,
    out_type=jax.ShapeDtypeStruct((2 * N, H), jnp.float32),
    scratch_types=[
        pltpu.VMEM_SHARED((N, H // 2), jnp.int32),
        pltpu.VMEM_SHARED((ACCR, H), jnp.float32),
        pltpu.VMEM((PH, CH), jnp.int32),
        pltpu.VMEM((PH, CH), jnp.int32),
        pltpu.VMEM((PH, CH), jnp.float32),
        pltpu.VMEM((CH, H // 2), jnp.int32),
        pltpu.VMEM((CH, H // 2), jnp.int32),
        pltpu.VMEM((CH, H), jnp.float32),
        pltpu.VMEM((CH, H), jnp.float32),
        pltpu.VMEM((CH,), jnp.int32),
        pltpu.VMEM((CH,), jnp.int32),
        pltpu.SemaphoreType.DMA,
        pltpu.SemaphoreType.DMA,
        pltpu.SemaphoreType.DMA,
        pltpu.SemaphoreType.DMA,
    ],
)
def _sc_propagate(h_hbm, src_hbm, dst_hbm, w_hbm, o_stack,
                  stage, acc, src_t, dst_t, w_t, gba, gbb, sba, sbb,
                  dla, dlb, sem_ga, sem_gb, sem_sa, sem_sb):
    c = lax.axis_index("c")
    s = lax.axis_index("s")
    half = c * N

    pltpu.sync_copy(h_hbm.at[pl.ds(half + s * RS, RS)],
                    stage.at[pl.ds(s * RS, RS)])

    @pl.when(s == 0)
    def _():
        pltpu.sync_copy(h_hbm.at[pl.ds(half + NT * RS, N - NT * RS)],
                        stage.at[pl.ds(NT * RS, N - NT * RS)])

    zv = jnp.zeros((16,), jnp.float32)

    def _zrow(j, carry):
        for k in range(H // 16):
            sba[j, pl.ds(k * 16, 16)] = zv
        return carry

    def _fire_gather(i, gbuf, sem):
        pass

    def _wait_gather(gbuf, sem):
        pass

    himask = jnp.full((16,), -65536, jnp.int32)

    def _scale(i, gbuf, sbuf):
        def _grp(q, inner):
            wvec = w_t[i, pl.ds(q * 16, 16)]

            def _one(j, inner2):
                wspl = wvec.at[jnp.full((16,), j, jnp.int32)].get(
                    mode="promise_in_bounds")
                r = q * 16 + j
                for k in range(H // 32):
                    ab = gbuf[r, pl.ds(16 * k, 16)]
                    a = lax.bitcast_convert_type(ab << 16, jnp.float32)
                    b = lax.bitcast_convert_type(ab & himask, jnp.float32)
                    sbuf[r, pl.ds(32 * k, 16)] = a * wspl
                    sbuf[r, pl.ds(32 * k + 16, 16)] = b * wspl
                return inner2

            return lax.fori_loop(0, 16, _one, inner)

        lax.fori_loop(0, CH // 16, _grp, 0)

    def _fire_scatter(sbuf, dbuf, sem):
        pass

    def _wait_scatter(sbuf, sem):
        pass

    for pass_ in range(2):
        nbase = pass_ * NP
        nb = jnp.full((16,), nbase, jnp.int32)
        npv = jnp.full((16,), NP, jnp.int32)

        lax.fori_loop(0, CH, _zrow, 0)
        for q in range(ZPT // CH):
            pltpu.sync_copy(sba, acc.at[pl.ds(s * ZPT + q * CH, CH)])
        pltpu.sync_copy(sba.at[pl.ds(0, ZPT - (ZPT // CH) * CH)],
                        acc.at[pl.ds(s * ZPT + (ZPT // CH) * CH,
                                     ZPT - (ZPT // CH) * CH)])

        @pl.when(s == 0)
        def _():
            pltpu.sync_copy(sba.at[pl.ds(0, ACCR - NT * ZPT)],
                            acc.at[pl.ds(NT * ZPT, ACCR - NT * ZPT)])

        plsc.subcore_barrier()

        def _dloc(i, dbuf):
            for q in range(CH // 16):
                sl = pl.ds(q * 16, 16)
                d = dst_t[i, sl] - nb
                ok = (d >= 0) & (d < npv)
                dbuf[sl] = jnp.where(ok, d, npv)

        def _pair(p, carry):
            i = 2 * p
            _wait_gather(gba, sem_ga)
            _scale(i, gba, sba)
            _dloc(i, dla)
            _fire_scatter(sba, dla, sem_sa)
            _wait_gather(gbb, sem_gb)
            _scale(i + 1, gbb, sbb)
            _dloc(i + 1, dlb)
            _fire_scatter(sbb, dlb, sem_sb)
            _wait_scatter(sba, sem_sa)
            _fire_gather(jnp.minimum(i + 2, PH - 1), gba, sem_ga)
            _wait_scatter(sbb, sem_sb)
            _fire_gather(jnp.minimum(i + 3, PH - 1), gbb, sem_gb)
            return carry

        def _phase(ph, carry):
            pltpu.sync_copy(src_hbm.at[s, pl.ds(ph * PH, PH)], src_t)
            pltpu.sync_copy(dst_hbm.at[s, pl.ds(ph * PH, PH)], dst_t)
            pltpu.sync_copy(w_hbm.at[s, pl.ds(ph * PH, PH)], w_t)
            _fire_gather(0, gba, sem_ga)
            _fire_gather(1, gbb, sem_gb)
            lax.fori_loop(0, PH // 2, _pair, 0)
            _wait_gather(gba, sem_ga)
            _wait_gather(gbb, sem_gb)
            return carry

        lax.fori_loop(0, NCHP // PH, _phase, 0)

        plsc.subcore_barrier()

        ob = half + nbase
        for q in range(ZPT // CH):
            pltpu.sync_copy(acc.at[pl.ds(s * ZPT + q * CH, CH)],
                            o_stack.at[pl.ds(ob + s * ZPT + q * CH, CH)])
        pltpu.sync_copy(acc.at[pl.ds(s * ZPT + (ZPT // CH) * CH,
                                     ZPT - (ZPT // CH) * CH)],
                        o_stack.at[pl.ds(ob + s * ZPT + (ZPT // CH) * CH,
                                         ZPT - (ZPT // CH) * CH)])

        @pl.when(s == 0)
        def _():
            pltpu.sync_copy(acc.at[pl.ds(NT * ZPT, NP - NT * ZPT)],
                            o_stack.at[pl.ds(ob + NT * ZPT, NP - NT * ZPT)])

        plsc.subcore_barrier()


BR = 2000
G = N // BR

_f32 = jnp.float32
_bf16 = jnp.bfloat16
_sds = jax.ShapeDtypeStruct


def _mm_first_body(x_ref, w_ref, b_ref, o_ref):
    o_ref[...] = (jnp.dot(x_ref[...], w_ref[...], preferred_element_type=_f32)
                  + b_ref[...])


def _mm_mid_body(lo_ref, hi_ref, w_ref, b_ref, o_ref):
    x = jnp.maximum(jnp.concatenate([lo_ref[...], hi_ref[...]], axis=1), 0.0)
    o_ref[...] = jnp.dot(x, w_ref[...], preferred_element_type=_f32) + b_ref[...]


def _pack_bf16(h):
    return lax.bitcast_convert_type(
        h.astype(_bf16).reshape(2 * N, H // 2, 2), jnp.int32)


def _proj_body(lo_ref, hi_ref, p1_ref, bp1_ref, p2_ref, bp2_ref, z_ref, h_ref):
    hcat = jnp.concatenate([lo_ref[...], hi_ref[...]], axis=1)
    h_ref[...] = hcat
    t = jnp.maximum(
        jnp.dot(hcat, p1_ref[...], preferred_element_type=_f32) + bp1_ref[...],
        0.0)
    z_ref[...] = jnp.dot(t, p2_ref[...], preferred_element_type=_f32) + bp2_ref[...]


_x_spec = pl.BlockSpec((BR, D), lambda i, j: (i, 0))
_whalf_spec = pl.BlockSpec((D, H), lambda i, j: (0, j))
_bhalf_spec = pl.BlockSpec((1, H), lambda i, j: (0, j))
_stack_out_spec = pl.BlockSpec((BR, H), lambda i, j: (j * G + i, 0))
_lo_spec = pl.BlockSpec((BR, H), lambda i, j: (i, 0))
_hi_spec = pl.BlockSpec((BR, H), lambda i, j: (G + i, 0))
_stack_sds = _sds((2 * N, H), _f32)


def _mm_first(x, W, b):
    return pl.pallas_call(
        _mm_first_body, grid=(G, 2),
        in_specs=[_x_spec, _whalf_spec, _bhalf_spec],
        out_specs=_stack_out_spec,
        out_shape=_stack_sds,
    )(x, W, b.reshape(1, D))


def _mm_mid(g, W, b):
    return pl.pallas_call(
        _mm_mid_body, grid=(G, 2),
        in_specs=[_lo_spec, _hi_spec, _whalf_spec, _bhalf_spec],
        out_specs=_stack_out_spec,
        out_shape=_stack_sds,
    )(g, g, W, b.reshape(1, D))


def _proj(g, P1, bp1, P2, bp2):
    row_spec = pl.BlockSpec((BR, D), lambda i: (i, 0))
    lo = pl.BlockSpec((BR, H), lambda i: (i, 0))
    hi = pl.BlockSpec((BR, H), lambda i: (G + i, 0))
    w_spec = pl.BlockSpec((D, D), lambda i: (0, 0))
    b_spec = pl.BlockSpec((1, D), lambda i: (0, 0))
    return pl.pallas_call(
        _proj_body, grid=(G,),
        in_specs=[lo, hi, w_spec, b_spec, w_spec, b_spec],
        out_specs=[row_spec, row_spec],
        out_shape=[_sds((N, D), _f32), _sds((N, D), _f32)],
    )(g, g, P1, bp1.reshape(1, D), P2, bp2.reshape(1, D))


def _unperm(g):
    return g.reshape(2 * N, H // 32, 2, 16).transpose(0, 1, 3, 2).reshape(
        2 * N, H)


def kernel(x, edge_index, edge_weight, W1, b1, W2, b2, W3, b3, P1, bp1, P2, bp2):
    pad = (0, NT * NCHP * CH - E)
    src = jnp.pad(edge_index[0].astype(jnp.int32), pad).reshape(NT, NCHP, CH)
    dst = jnp.pad(edge_index[1].astype(jnp.int32), pad).reshape(NT, NCHP, CH)
    w = jnp.pad(edge_weight.astype(jnp.float32), pad).reshape(NT, NCHP, CH)

    h = _mm_first(x, W1, b1)
    g = _sc_propagate(_pack_bf16(h), src, dst, w)
    h = _mm_mid(g, W2[_PERM256], b2)
    g = _sc_propagate(_pack_bf16(h), src, dst, w)
    h = _mm_mid(g, W3[_PERM256], b3)
    g = _sc_propagate(_pack_bf16(h), src, dst, w)
    z, hout = _proj(_unperm(g), P1, bp1, P2, bp2)
    return (z, hout)


# speedup vs baseline: 1.6287x; 1.6287x over previous
import functools

import numpy as np

import jax
import jax.numpy as jnp
from jax import lax
from jax.experimental import pallas as pl
from jax.experimental.pallas import tpu as pltpu
from jax.experimental.pallas import tpu_sc as plsc

N = 10000
E = 160000
D = 256
H = 128
NT = 16
EPT = E // NT
CH = 64
NCHP = 160
PH = 16
ZPT = 624
REM = N - NT * ZPT

_mesh = plsc.VectorSubcoreMesh(core_axis_name="c", subcore_axis_name="s")


@functools.partial(
    pl.kernel,
    mesh=_mesh,
    out_type=jax.ShapeDtypeStruct((2 * N, H), jnp.float32),
    scratch_types=[
        pltpu.VMEM_SHARED((N, H), jnp.float32),
        pltpu.VMEM((PH, CH), jnp.int32),
        pltpu.VMEM((PH, CH), jnp.int32),
        pltpu.VMEM((PH, CH), jnp.float32),
        pltpu.VMEM((CH, H), jnp.float32),
        pltpu.VMEM((CH, H), jnp.float32),
        pltpu.SemaphoreType.DMA,
        pltpu.SemaphoreType.DMA,
        pltpu.SemaphoreType.DMA,
        pltpu.SemaphoreType.DMA,
    ],
)
def _sc_propagate(h_hbm, src_hbm, dst_hbm, w_hbm, o_stack,
                  acc, src_t, dst_t, w_t, gba, gbb,
                  sem_ga, sem_gb, sem_sa, sem_sb):
    c = lax.axis_index("c")
    s = lax.axis_index("s")
    half = c * N

    zv = jnp.zeros((16,), jnp.float32)

    def _zrow(j, carry):
        for k in range(H // 16):
            gba[j, pl.ds(k * 16, 16)] = zv
        return carry

    def _fire_gather(i, gbuf, sem):
        pltpu.async_copy(h_hbm.at[src_t.at[i]], gbuf, sem)

    def _wait_gather(gbuf, sem):
        pltpu.make_async_copy(h_hbm.at[pl.ds(0, CH)], gbuf, sem).wait()

    def _scale(i, gbuf):
        def _grp(q, inner):
            wvec = w_t[i, pl.ds(q * 16, 16)]

            def _one(j, inner2):
                wspl = wvec.at[jnp.full((16,), j, jnp.int32)].get(
                    mode="promise_in_bounds")
                r = q * 16 + j
                for k in range(H // 16):
                    sl = pl.ds(16 * k, 16)
                    gbuf[r, sl] = gbuf[r, sl] * wspl
                return inner2

            return lax.fori_loop(0, 16, _one, inner)

        lax.fori_loop(0, CH // 16, _grp, 0)

    def _fire_scatter(sbuf, dbuf, sem):
        pltpu.async_copy(sbuf, acc.at[dbuf], sem, add=True)

    def _wait_scatter(sbuf, sem):
        pltpu.make_async_copy(sbuf, acc.at[pl.ds(0, CH)], sem).wait()

    lax.fori_loop(0, CH, _zrow, 0)
    for q in range(ZPT // CH):
        pltpu.sync_copy(gba, acc.at[pl.ds(s * ZPT + q * CH, CH)])
    pltpu.sync_copy(gba.at[pl.ds(0, ZPT - (ZPT // CH) * CH)],
                    acc.at[pl.ds(s * ZPT + (ZPT // CH) * CH,
                                 ZPT - (ZPT // CH) * CH)])

    @pl.when(s == 0)
    def _():
        pltpu.sync_copy(gba.at[pl.ds(0, REM)], acc.at[pl.ds(NT * ZPT, REM)])

    plsc.subcore_barrier()

    hv = jnp.full((16,), half, jnp.int32)

    def _adj(j, carry):
        for q in range(CH // 16):
            sl = pl.ds(q * 16, 16)
            src_t[j, sl] = src_t[j, sl] + hv
        return carry

    def _pair(p, carry):
        i = 2 * p
        _wait_gather(gba, sem_ga)
        _scale(i, gba)
        _fire_scatter(gba, dst_t.at[i], sem_sa)
        _wait_gather(gbb, sem_gb)
        _scale(i + 1, gbb)
        _fire_scatter(gbb, dst_t.at[i + 1], sem_sb)
        _wait_scatter(gba, sem_sa)
        _fire_gather(jnp.minimum(i + 2, PH - 1), gba, sem_ga)
        _wait_scatter(gbb, sem_sb)
        _fire_gather(jnp.minimum(i + 3, PH - 1), gbb, sem_gb)
        return carry

    def _phase(ph, carry):
        pltpu.sync_copy(src_hbm.at[s, pl.ds(ph * PH, PH)], src_t)
        pltpu.sync_copy(dst_hbm.at[s, pl.ds(ph * PH, PH)], dst_t)
        pltpu.sync_copy(w_hbm.at[s, pl.ds(ph * PH, PH)], w_t)
        lax.fori_loop(0, PH, _adj, 0)
        _fire_gather(0, gba, sem_ga)
        _fire_gather(1, gbb, sem_gb)
        lax.fori_loop(0, PH // 2, _pair, 0)
        _wait_gather(gba, sem_ga)
        _wait_gather(gbb, sem_gb)
        return carry

    lax.fori_loop(0, NCHP // PH, _phase, 0)

    plsc.subcore_barrier()

    for q in range(ZPT // CH):
        pltpu.sync_copy(acc.at[pl.ds(s * ZPT + q * CH, CH)],
                        o_stack.at[pl.ds(half + s * ZPT + q * CH, CH)])
    pltpu.sync_copy(acc.at[pl.ds(s * ZPT + (ZPT // CH) * CH,
                                 ZPT - (ZPT // CH) * CH)],
                    o_stack.at[pl.ds(half + s * ZPT + (ZPT // CH) * CH,
                                     ZPT - (ZPT // CH) * CH)])

    @pl.when(s == 0)
    def _():
        pltpu.sync_copy(acc.at[pl.ds(NT * ZPT, REM)],
                        o_stack.at[pl.ds(half + NT * ZPT, REM)])


BR = 2000
G = N // BR

_f32 = jnp.float32
_bf16 = jnp.bfloat16
_sds = jax.ShapeDtypeStruct


def _mm_first_body(x_ref, w_ref, b_ref, o_ref):
    o_ref[...] = (jnp.dot(x_ref[...], w_ref[...], preferred_element_type=_f32)
                  + b_ref[...])


def _mm_mid_body(lo_ref, hi_ref, w_ref, b_ref, o_ref):
    x = jnp.maximum(jnp.concatenate([lo_ref[...], hi_ref[...]], axis=1), 0.0)
    o_ref[...] = jnp.dot(x, w_ref[...], preferred_element_type=_f32) + b_ref[...]


def _proj_body(lo_ref, hi_ref, p1_ref, bp1_ref, p2_ref, bp2_ref, z_ref, h_ref):
    hcat = jnp.concatenate([lo_ref[...], hi_ref[...]], axis=1)
    h_ref[...] = hcat
    t = jnp.maximum(
        jnp.dot(hcat, p1_ref[...], preferred_element_type=_f32) + bp1_ref[...],
        0.0)
    z_ref[...] = jnp.dot(t, p2_ref[...], preferred_element_type=_f32) + bp2_ref[...]


_x_spec = pl.BlockSpec((BR, D), lambda i, j: (i, 0))
_whalf_spec = pl.BlockSpec((D, H), lambda i, j: (0, j))
_bhalf_spec = pl.BlockSpec((1, H), lambda i, j: (0, j))
_stack_out_spec = pl.BlockSpec((BR, H), lambda i, j: (j * G + i, 0))
_lo_spec = pl.BlockSpec((BR, H), lambda i, j: (i, 0))
_hi_spec = pl.BlockSpec((BR, H), lambda i, j: (G + i, 0))
_stack_sds = _sds((2 * N, H), _f32)


def _mm_first(x, W, b):
    return pl.pallas_call(
        _mm_first_body, grid=(G, 2),
        in_specs=[_x_spec, _whalf_spec, _bhalf_spec],
        out_specs=_stack_out_spec,
        out_shape=_stack_sds,
    )(x, W, b.reshape(1, D))


def _mm_mid(g, W, b):
    return pl.pallas_call(
        _mm_mid_body, grid=(G, 2),
        in_specs=[_lo_spec, _hi_spec, _whalf_spec, _bhalf_spec],
        out_specs=_stack_out_spec,
        out_shape=_stack_sds,
    )(g, g, W, b.reshape(1, D))


def _proj(g, P1, bp1, P2, bp2):
    row_spec = pl.BlockSpec((BR, D), lambda i: (i, 0))
    lo = pl.BlockSpec((BR, H), lambda i: (i, 0))
    hi = pl.BlockSpec((BR, H), lambda i: (G + i, 0))
    w_spec = pl.BlockSpec((D, D), lambda i: (0, 0))
    b_spec = pl.BlockSpec((1, D), lambda i: (0, 0))
    return pl.pallas_call(
        _proj_body, grid=(G,),
        in_specs=[lo, hi, w_spec, b_spec, w_spec, b_spec],
        out_specs=[row_spec, row_spec],
        out_shape=[_sds((N, D), _f32), _sds((N, D), _f32)],
    )(g, g, P1, bp1.reshape(1, D), P2, bp2.reshape(1, D))


def kernel(x, edge_index, edge_weight, W1, b1, W2, b2, W3, b3, P1, bp1, P2, bp2):
    pad = (0, NT * NCHP * CH - E)
    src = jnp.pad(edge_index[0].astype(jnp.int32), pad).reshape(NT, NCHP, CH)
    dst = jnp.pad(edge_index[1].astype(jnp.int32), pad).reshape(NT, NCHP, CH)
    w = jnp.pad(edge_weight.astype(jnp.float32), pad).reshape(NT, NCHP, CH)

    h = _mm_first(x, W1, b1)
    g = _sc_propagate(h, src, dst, w)
    h = _mm_mid(g, W2, b2)
    g = _sc_propagate(h, src, dst, w)
    h = _mm_mid(g, W3, b3)
    g = _sc_propagate(h, src, dst, w)
    z, hout = _proj(g, P1, bp1, P2, bp2)
    return (z, hout)


# speedup vs baseline: 1.7883x; 1.0980x over previous
import functools

import numpy as np

import jax
import jax.numpy as jnp
from jax import lax
from jax.experimental import pallas as pl
from jax.experimental.pallas import tpu as pltpu
from jax.experimental.pallas import tpu_sc as plsc

N = 10000
E = 160000
D = 256
H = 128
NT = 16
EPT = E // NT
CH = 128
NCHP = 80
PH = 16
ZPT = 624
REM = N - NT * ZPT

_mesh = plsc.VectorSubcoreMesh(core_axis_name="c", subcore_axis_name="s")


@functools.partial(
    pl.kernel,
    mesh=_mesh,
    out_type=jax.ShapeDtypeStruct((2 * N, H), jnp.float32),
    scratch_types=[
        pltpu.VMEM_SHARED((N, H), jnp.float32),
        pltpu.VMEM((PH, CH), jnp.int32),
        pltpu.VMEM((PH, CH), jnp.int32),
        pltpu.VMEM((PH, CH), jnp.float32),
        pltpu.VMEM((CH, H), jnp.float32),
        pltpu.VMEM((CH, H), jnp.float32),
        pltpu.SemaphoreType.DMA,
        pltpu.SemaphoreType.DMA,
        pltpu.SemaphoreType.DMA,
        pltpu.SemaphoreType.DMA,
    ],
)
def _sc_propagate(h_hbm, src_hbm, dst_hbm, w_hbm, o_stack,
                  acc, src_t, dst_t, w_t, gba, gbb,
                  sem_ga, sem_gb, sem_sa, sem_sb):
    c = lax.axis_index("c")
    s = lax.axis_index("s")
    half = c * N

    zv = jnp.zeros((16,), jnp.float32)

    def _zrow(j, carry):
        for k in range(H // 16):
            gba[j, pl.ds(k * 16, 16)] = zv
        return carry

    def _fire_gather(i, gbuf, sem):
        pltpu.async_copy(h_hbm.at[src_t.at[i]], gbuf, sem)

    def _wait_gather(gbuf, sem):
        pltpu.make_async_copy(h_hbm.at[pl.ds(0, CH)], gbuf, sem).wait()

    def _scale(i, gbuf):
        def _grp(q, inner):
            wvec = w_t[i, pl.ds(q * 16, 16)]

            def _one(j, inner2):
                wspl = wvec.at[jnp.full((16,), j, jnp.int32)].get(
                    mode="promise_in_bounds")
                r = q * 16 + j
                for k in range(H // 16):
                    sl = pl.ds(16 * k, 16)
                    gbuf[r, sl] = gbuf[r, sl] * wspl
                return inner2

            return lax.fori_loop(0, 16, _one, inner)

        lax.fori_loop(0, CH // 16, _grp, 0)

    def _fire_scatter(sbuf, dbuf, sem):
        pltpu.async_copy(sbuf, acc.at[dbuf], sem, add=True)

    def _wait_scatter(sbuf, sem):
        pltpu.make_async_copy(sbuf, acc.at[pl.ds(0, CH)], sem).wait()

    lax.fori_loop(0, CH, _zrow, 0)
    for q in range(ZPT // CH):
        pltpu.sync_copy(gba, acc.at[pl.ds(s * ZPT + q * CH, CH)])
    pltpu.sync_copy(gba.at[pl.ds(0, ZPT - (ZPT // CH) * CH)],
                    acc.at[pl.ds(s * ZPT + (ZPT // CH) * CH,
                                 ZPT - (ZPT // CH) * CH)])

    @pl.when(s == 0)
    def _():
        pltpu.sync_copy(gba.at[pl.ds(0, REM)], acc.at[pl.ds(NT * ZPT, REM)])

    plsc.subcore_barrier()

    hv = jnp.full((16,), half, jnp.int32)

    def _adj(j, carry):
        for q in range(CH // 16):
            sl = pl.ds(q * 16, 16)
            src_t[j, sl] = src_t[j, sl] + hv
        return carry

    def _pair(p, carry):
        i = 2 * p
        _wait_gather(gba, sem_ga)
        _scale(i, gba)
        _fire_scatter(gba, dst_t.at[i], sem_sa)
        _wait_gather(gbb, sem_gb)
        _scale(i + 1, gbb)
        _fire_scatter(gbb, dst_t.at[i + 1], sem_sb)
        _wait_scatter(gba, sem_sa)
        _fire_gather(jnp.minimum(i + 2, PH - 1), gba, sem_ga)
        _wait_scatter(gbb, sem_sb)
        _fire_gather(jnp.minimum(i + 3, PH - 1), gbb, sem_gb)
        return carry

    def _phase(ph, carry):
        pltpu.sync_copy(src_hbm.at[s, pl.ds(ph * PH, PH)], src_t)
        pltpu.sync_copy(dst_hbm.at[s, pl.ds(ph * PH, PH)], dst_t)
        pltpu.sync_copy(w_hbm.at[s, pl.ds(ph * PH, PH)], w_t)
        lax.fori_loop(0, PH, _adj, 0)
        _fire_gather(0, gba, sem_ga)
        _fire_gather(1, gbb, sem_gb)
        lax.fori_loop(0, PH // 2, _pair, 0)
        _wait_gather(gba, sem_ga)
        _wait_gather(gbb, sem_gb)
        return carry

    lax.fori_loop(0, NCHP // PH, _phase, 0)

    plsc.subcore_barrier()

    for q in range(ZPT // CH):
        pltpu.sync_copy(acc.at[pl.ds(s * ZPT + q * CH, CH)],
                        o_stack.at[pl.ds(half + s * ZPT + q * CH, CH)])
    pltpu.sync_copy(acc.at[pl.ds(s * ZPT + (ZPT // CH) * CH,
                                 ZPT - (ZPT // CH) * CH)],
                    o_stack.at[pl.ds(half + s * ZPT + (ZPT // CH) * CH,
                                     ZPT - (ZPT // CH) * CH)])

    @pl.when(s == 0)
    def _():
        pltpu.sync_copy(acc.at[pl.ds(NT * ZPT, REM)],
                        o_stack.at[pl.ds(half + NT * ZPT, REM)])


BR = 2000
G = N // BR

_f32 = jnp.float32
_bf16 = jnp.bfloat16
_sds = jax.ShapeDtypeStruct


def _mm_first_body(x_ref, w_ref, b_ref, o_ref):
    o_ref[...] = (jnp.dot(x_ref[...], w_ref[...], preferred_element_type=_f32)
                  + b_ref[...])


def _mm_mid_body(lo_ref, hi_ref, w_ref, b_ref, o_ref):
    x = jnp.maximum(jnp.concatenate([lo_ref[...], hi_ref[...]], axis=1), 0.0)
    o_ref[...] = jnp.dot(x, w_ref[...], preferred_element_type=_f32) + b_ref[...]


def _proj_body(lo_ref, hi_ref, p1_ref, bp1_ref, p2_ref, bp2_ref, z_ref, h_ref):
    hcat = jnp.concatenate([lo_ref[...], hi_ref[...]], axis=1)
    h_ref[...] = hcat
    t = jnp.maximum(
        jnp.dot(hcat, p1_ref[...], preferred_element_type=_f32) + bp1_ref[...],
        0.0)
    z_ref[...] = jnp.dot(t, p2_ref[...], preferred_element_type=_f32) + bp2_ref[...]


_x_spec = pl.BlockSpec((BR, D), lambda i, j: (i, 0))
_whalf_spec = pl.BlockSpec((D, H), lambda i, j: (0, j))
_bhalf_spec = pl.BlockSpec((1, H), lambda i, j: (0, j))
_stack_out_spec = pl.BlockSpec((BR, H), lambda i, j: (j * G + i, 0))
_lo_spec = pl.BlockSpec((BR, H), lambda i, j: (i, 0))
_hi_spec = pl.BlockSpec((BR, H), lambda i, j: (G + i, 0))
_stack_sds = _sds((2 * N, H), _f32)


def _mm_first(x, W, b):
    return pl.pallas_call(
        _mm_first_body, grid=(G, 2),
        in_specs=[_x_spec, _whalf_spec, _bhalf_spec],
        out_specs=_stack_out_spec,
        out_shape=_stack_sds,
    )(x, W, b.reshape(1, D))


def _mm_mid(g, W, b):
    return pl.pallas_call(
        _mm_mid_body, grid=(G, 2),
        in_specs=[_lo_spec, _hi_spec, _whalf_spec, _bhalf_spec],
        out_specs=_stack_out_spec,
        out_shape=_stack_sds,
    )(g, g, W, b.reshape(1, D))


def _proj(g, P1, bp1, P2, bp2):
    row_spec = pl.BlockSpec((BR, D), lambda i: (i, 0))
    lo = pl.BlockSpec((BR, H), lambda i: (i, 0))
    hi = pl.BlockSpec((BR, H), lambda i: (G + i, 0))
    w_spec = pl.BlockSpec((D, D), lambda i: (0, 0))
    b_spec = pl.BlockSpec((1, D), lambda i: (0, 0))
    return pl.pallas_call(
        _proj_body, grid=(G,),
        in_specs=[lo, hi, w_spec, b_spec, w_spec, b_spec],
        out_specs=[row_spec, row_spec],
        out_shape=[_sds((N, D), _f32), _sds((N, D), _f32)],
    )(g, g, P1, bp1.reshape(1, D), P2, bp2.reshape(1, D))


def kernel(x, edge_index, edge_weight, W1, b1, W2, b2, W3, b3, P1, bp1, P2, bp2):
    pad = (0, NT * NCHP * CH - E)
    src = jnp.pad(edge_index[0].astype(jnp.int32), pad).reshape(NT, NCHP, CH)
    dst = jnp.pad(edge_index[1].astype(jnp.int32), pad).reshape(NT, NCHP, CH)
    w = jnp.pad(edge_weight.astype(jnp.float32), pad).reshape(NT, NCHP, CH)

    h = _mm_first(x, W1, b1)
    g = _sc_propagate(h, src, dst, w)
    h = _mm_mid(g, W2, b2)
    g = _sc_propagate(h, src, dst, w)
    h = _mm_mid(g, W3, b3)
    g = _sc_propagate(h, src, dst, w)
    z, hout = _proj(g, P1, bp1, P2, bp2)
    return (z, hout)
